# baseline (device time: 122187 ns/iter reference)
import jax
import jax.numpy as jnp
from jax import lax
from jax.experimental import pallas as pl
from jax.experimental.pallas import tpu as pltpu

N_DEV = 16
BLK = 64


def kernel(x, Wq, K_ext, V_ext, Wo):
    B, S, Dm = x.shape
    _, _, H, Dh = K_ext.shape
    HD = H * Dh

    xb = x.astype(jnp.bfloat16)
    wqb = Wq.astype(jnp.bfloat16)
    wob = Wo.astype(jnp.bfloat16)
    kv = jnp.concatenate(
        [K_ext.reshape(B, S, HD), V_ext.reshape(B, S, HD)], axis=0
    ).astype(jnp.bfloat16)

    def body(x_ref, wq_ref, kv_ref, wo_ref, out_ref,
             kv_all, local_sem, send_sems, recv_sems):
        my = lax.axis_index("i")
        left = (my + N_DEV - 1) % N_DEV
        right = (my + 1) % N_DEV

        cp = pltpu.make_async_copy(kv_ref, kv_all.at[my], local_sem)
        cp.start()

        barrier_sem = pltpu.get_barrier_semaphore()
        for nbr in (left, right):
            pl.semaphore_signal(barrier_sem, inc=1, device_id=(nbr,),
                                device_id_type=pl.DeviceIdType.MESH)
        pl.semaphore_wait(barrier_sem, 2)

        qs = []
        for b in range(B):
            q = lax.dot_general(x_ref[b], wq_ref[...],
                                (((1,), (0,)), ((), ())),
                                preferred_element_type=jnp.float32)
            qs.append(q.astype(jnp.bfloat16))

        cp.wait()

        qi = lax.broadcasted_iota(jnp.int32, (S, S), 0)
        kj = lax.broadcasted_iota(jnp.int32, (S, S), 1)
        qb = my * (S // BLK) + qi // BLK

        NEG = jnp.float32(-1e9)
        nbh = B * H
        m_st = [jnp.full((S, 1), -1e30, jnp.float32) for _ in range(nbh)]
        l_st = [jnp.zeros((S, 1), jnp.float32) for _ in range(nbh)]
        a_st = [jnp.zeros((S, Dh), jnp.float32) for _ in range(nbh)]

        for hop in range(N_DEV):
            c = (my + N_DEV - hop) % N_DEV
            if hop < N_DEV - 1:
                rdma = pltpu.make_async_remote_copy(
                    src_ref=kv_all.at[c], dst_ref=kv_all.at[c],
                    send_sem=send_sems.at[hop], recv_sem=recv_sems.at[hop],
                    device_id=(right,), device_id_type=pl.DeviceIdType.MESH)
                rdma.start()

            kb = c * (S // BLK) + kj // BLK
            mask = (qb == kb) | (kb == 0) | ((qb + kb) % 3 == 0)

            for b in range(B):
                kmat = kv_all[c, b]
                vmat = kv_all[c, B + b]
                for hh in range(H):
                    i = b * H + hh
                    qh = qs[b][:, hh * Dh:(hh + 1) * Dh]
                    kh = kmat[:, hh * Dh:(hh + 1) * Dh]
                    vh = vmat[:, hh * Dh:(hh + 1) * Dh]
                    s = lax.dot_general(qh, kh, (((1,), (1,)), ((), ())),
                                        preferred_element_type=jnp.float32)
                    s = jnp.where(mask, s * 0.125, NEG)
                    m_new = jnp.maximum(m_st[i],
                                        jnp.max(s, axis=1, keepdims=True))
                    corr = jnp.exp(m_st[i] - m_new)
                    p = jnp.exp(s - m_new)
                    l_st[i] = l_st[i] * corr + jnp.sum(p, axis=1,
                                                       keepdims=True)
                    pv = lax.dot_general(p.astype(jnp.bfloat16), vh,
                                         (((1,), (0,)), ((), ())),
                                         preferred_element_type=jnp.float32)
                    a_st[i] = a_st[i] * corr + pv
                    m_st[i] = m_new

            if hop < N_DEV - 1:
                rdma.wait()

        for b in range(B):
            ctx = jnp.concatenate(
                [(a_st[b * H + hh] / l_st[b * H + hh]).astype(jnp.bfloat16)
                 for hh in range(H)], axis=1)
            out_ref[b] = lax.dot_general(ctx, wo_ref[...],
                                         (((1,), (0,)), ((), ())),
                                         preferred_element_type=jnp.float32)

    return pl.pallas_call(
        body,
        out_shape=jax.ShapeDtypeStruct((B, S, Dm), jnp.float32),
        in_specs=[pl.BlockSpec(memory_space=pltpu.VMEM)] * 4,
        out_specs=pl.BlockSpec(memory_space=pltpu.VMEM),
        scratch_shapes=[
            pltpu.VMEM((N_DEV, 2 * B, S, HD), jnp.bfloat16),
            pltpu.SemaphoreType.DMA,
            pltpu.SemaphoreType.DMA((N_DEV - 1,)),
            pltpu.SemaphoreType.DMA((N_DEV - 1,)),
        ],
        compiler_params=pltpu.CompilerParams(collective_id=0),
    )(xb, wqb, kv, wob)


# device time: 79113 ns/iter; 1.5445x vs baseline; 1.5445x over previous
import jax
import jax.numpy as jnp
from jax import lax
from jax.experimental import pallas as pl
from jax.experimental.pallas import tpu as pltpu

N_DEV = 16
BLK = 64
R_HOPS = N_DEV // 2
L_HOPS = N_DEV - 1 - R_HOPS


def kernel(x, Wq, K_ext, V_ext, Wo):
    B, S, Dm = x.shape
    _, _, H, Dh = K_ext.shape
    HD = H * Dh

    xb = x.astype(jnp.bfloat16)
    wqb = Wq.astype(jnp.bfloat16)
    wob = Wo.astype(jnp.bfloat16)
    kv = jnp.concatenate(
        [K_ext.reshape(B, S, HD), V_ext.reshape(B, S, HD)], axis=0
    ).astype(jnp.bfloat16)

    def body(x_ref, wq_ref, kv_ref, wo_ref, out_ref,
             kv_all, local_sem, send_r, recv_r, send_l, recv_l):
        my = lax.axis_index("i")
        left = (my + N_DEV - 1) % N_DEV
        right = (my + 1) % N_DEV

        cp = pltpu.make_async_copy(kv_ref, kv_all.at[my], local_sem)
        cp.start()

        qs = []
        for b in range(B):
            q = lax.dot_general(x_ref[b], wq_ref[...],
                                (((1,), (0,)), ((), ())),
                                preferred_element_type=jnp.float32)
            qs.append((q * 0.125).astype(jnp.bfloat16))

        barrier_sem = pltpu.get_barrier_semaphore()
        for nbr in (left, right):
            pl.semaphore_signal(barrier_sem, inc=1, device_id=(nbr,),
                                device_id_type=pl.DeviceIdType.MESH)
        pl.semaphore_wait(barrier_sem, 2)

        cp.wait()

        qi = lax.broadcasted_iota(jnp.int32, (S, S), 0)
        kj = lax.broadcasted_iota(jnp.int32, (S, S), 1)
        qb = my * (S // BLK) + qi // BLK
        kjb = kj // BLK

        NEG = jnp.float32(-1e9)
        nbh = B * H
        l_st = [jnp.zeros((S, 1), jnp.float32) for _ in range(nbh)]
        a_st = [jnp.zeros((S, Dh), jnp.float32) for _ in range(nbh)]

        def consume(c):
            kb = c * (S // BLK) + kjb
            keep = (qb == kb) | (kb == 0) | ((qb + kb) % 3 == 0)
            bias = jnp.where(keep, 0.0, NEG)
            for b in range(B):
                kmat = kv_all[c, b]
                vmat = kv_all[c, B + b]
                for hh in range(H):
                    i = b * H + hh
                    qh = qs[b][:, hh * Dh:(hh + 1) * Dh]
                    kh = kmat[:, hh * Dh:(hh + 1) * Dh]
                    vh = vmat[:, hh * Dh:(hh + 1) * Dh]
                    s = lax.dot_general(qh, kh, (((1,), (1,)), ((), ())),
                                        preferred_element_type=jnp.float32)
                    p = jnp.exp(s + bias)
                    l_st[i] += jnp.sum(p, axis=1, keepdims=True)
                    a_st[i] += lax.dot_general(
                        p.astype(jnp.bfloat16), vh, (((1,), (0,)), ((), ())),
                        preferred_element_type=jnp.float32)

        for h in range(R_HOPS + 1):
            if h < R_HOPS:
                c_r = (my + N_DEV - h) % N_DEV
                rdma_r = pltpu.make_async_remote_copy(
                    src_ref=kv_all.at[c_r], dst_ref=kv_all.at[c_r],
                    send_sem=send_r.at[h], recv_sem=recv_r.at[h],
                    device_id=(right,), device_id_type=pl.DeviceIdType.MESH)
                rdma_r.start()
            if h < L_HOPS:
                c_l = (my + h) % N_DEV
                rdma_l = pltpu.make_async_remote_copy(
                    src_ref=kv_all.at[c_l], dst_ref=kv_all.at[c_l],
                    send_sem=send_l.at[h], recv_sem=recv_l.at[h],
                    device_id=(left,), device_id_type=pl.DeviceIdType.MESH)
                rdma_l.start()

            if h == 0:
                consume(my)
            else:
                consume((my + N_DEV - h) % N_DEV)
                if h <= L_HOPS:
                    consume((my + h) % N_DEV)

            if h < R_HOPS:
                rdma_r.wait()
            if h < L_HOPS:
                rdma_l.wait()

        for b in range(B):
            ctx = jnp.concatenate(
                [(a_st[b * H + hh] / l_st[b * H + hh]).astype(jnp.bfloat16)
                 for hh in range(H)], axis=1)
            out_ref[b] = lax.dot_general(ctx, wo_ref[...],
                                         (((1,), (0,)), ((), ())),
                                         preferred_element_type=jnp.float32)

    return pl.pallas_call(
        body,
        out_shape=jax.ShapeDtypeStruct((B, S, Dm), jnp.float32),
        in_specs=[pl.BlockSpec(memory_space=pltpu.VMEM)] * 4,
        out_specs=pl.BlockSpec(memory_space=pltpu.VMEM),
        scratch_shapes=[
            pltpu.VMEM((N_DEV, 2 * B, S, HD), jnp.bfloat16),
            pltpu.SemaphoreType.DMA,
            pltpu.SemaphoreType.DMA((R_HOPS,)),
            pltpu.SemaphoreType.DMA((R_HOPS,)),
            pltpu.SemaphoreType.DMA((L_HOPS,)),
            pltpu.SemaphoreType.DMA((L_HOPS,)),
        ],
        compiler_params=pltpu.CompilerParams(collective_id=0),
    )(xb, wqb, kv, wob)


# device time: 55564 ns/iter; 2.1990x vs baseline; 1.4238x over previous
import jax
import jax.numpy as jnp
from jax import lax
from jax.experimental import pallas as pl
from jax.experimental.pallas import tpu as pltpu

N_DEV = 16
BLK = 64
R_HOPS = N_DEV // 2
L_HOPS = N_DEV - 1 - R_HOPS

PERM = [0, 4, 8, 12, 13, 9, 5, 1, 2, 6, 10, 14, 15, 11, 7, 3]
INV = [0] * N_DEV
for _r, _l in enumerate(PERM):
    INV[_l] = _r


def _lut(idx, table):
    out = jnp.int32(table[0])
    for j in range(1, len(table)):
        out = jnp.where(idx == j, jnp.int32(table[j]), out)
    return out


def kernel(x, Wq, K_ext, V_ext, Wo):
    B, S, Dm = x.shape
    _, _, H, Dh = K_ext.shape
    HD = H * Dh

    xb = x.astype(jnp.bfloat16)
    wqb = Wq.astype(jnp.bfloat16)
    wob = Wo.astype(jnp.bfloat16)
    kv = jnp.concatenate(
        [K_ext.reshape(B, S, HD), V_ext.reshape(B, S, HD)], axis=0
    ).astype(jnp.bfloat16)

    def body(x_ref, wq_ref, kv_ref, wo_ref, out_ref,
             kv_all, local_sem, send_r, recv_r, send_l, recv_l):
        my = lax.axis_index("i")
        pos = _lut(my, INV)
        right = _lut((pos + 1) % N_DEV, PERM)
        left = _lut((pos + N_DEV - 1) % N_DEV, PERM)

        cp = pltpu.make_async_copy(kv_ref, kv_all.at[my], local_sem)
        cp.start()

        qs = []
        for b in range(B):
            q = lax.dot_general(x_ref[b], wq_ref[...],
                                (((1,), (0,)), ((), ())),
                                preferred_element_type=jnp.float32)
            qs.append((q * 0.125).astype(jnp.bfloat16))

        barrier_sem = pltpu.get_barrier_semaphore()
        for nbr in (left, right):
            pl.semaphore_signal(barrier_sem, inc=1, device_id=(nbr,),
                                device_id_type=pl.DeviceIdType.MESH)
        pl.semaphore_wait(barrier_sem, 2)

        cp.wait()

        qi = lax.broadcasted_iota(jnp.int32, (S, S), 0)
        kj = lax.broadcasted_iota(jnp.int32, (S, S), 1)
        qb = my * (S // BLK) + qi // BLK
        kjb = kj // BLK

        NEG = jnp.float32(-1e9)
        nbh = B * H
        l_st = [jnp.zeros((S, 1), jnp.float32) for _ in range(nbh)]
        a_st = [jnp.zeros((S, Dh), jnp.float32) for _ in range(nbh)]

        def consume(c):
            kb = c * (S // BLK) + kjb
            keep = (qb == kb) | (kb == 0) | ((qb + kb) % 3 == 0)
            bias = jnp.where(keep, 0.0, NEG)
            for b in range(B):
                kmat = kv_all[c, b]
                vmat = kv_all[c, B + b]
                for hh in range(H):
                    i = b * H + hh
                    qh = qs[b][:, hh * Dh:(hh + 1) * Dh]
                    kh = kmat[:, hh * Dh:(hh + 1) * Dh]
                    vh = vmat[:, hh * Dh:(hh + 1) * Dh]
                    s = lax.dot_general(qh, kh, (((1,), (1,)), ((), ())),
                                        preferred_element_type=jnp.float32)
                    p = jnp.exp(s + bias)
                    l_st[i] += jnp.sum(p, axis=1, keepdims=True)
                    a_st[i] += lax.dot_general(
                        p.astype(jnp.bfloat16), vh, (((1,), (0,)), ((), ())),
                        preferred_element_type=jnp.float32)

        def rdma(c, half, hop, sems_s, sems_r, dev):
            sl = pl.ds(half * B, B)
            return pltpu.make_async_remote_copy(
                src_ref=kv_all.at[c, sl], dst_ref=kv_all.at[c, sl],
                send_sem=sems_s.at[hop, half], recv_sem=sems_r.at[hop, half],
                device_id=(dev,), device_id_type=pl.DeviceIdType.MESH)

        for half in (0, 1):
            rdma(my, half, 0, send_r, recv_r, right).start()
        for half in (0, 1):
            rdma(my, half, 0, send_l, recv_l, left).start()
        consume(my)

        for h in range(1, R_HOPS + 1):
            c_r = _lut((pos + N_DEV - h) % N_DEV, PERM)
            c_l = _lut((pos + h) % N_DEV, PERM)
            for half in (0, 1):
                rdma(c_r, half, h - 1, send_r, recv_r, right).wait_recv()
                if h < R_HOPS:
                    rdma(c_r, half, h, send_r, recv_r, right).start()
                if h - 1 < L_HOPS:
                    rdma(c_l, half, h - 1, send_l, recv_l, left).wait_recv()
                    if h < L_HOPS:
                        rdma(c_l, half, h, send_l, recv_l, left).start()
            consume(c_r)
            if h <= L_HOPS:
                consume(c_l)

        for h in range(R_HOPS):
            c_r = _lut((pos + N_DEV - h) % N_DEV, PERM)
            for half in (0, 1):
                rdma(c_r, half, h, send_r, recv_r, right).wait_send()
        for h in range(L_HOPS):
            c_l = _lut((pos + h) % N_DEV, PERM)
            for half in (0, 1):
                rdma(c_l, half, h, send_l, recv_l, left).wait_send()

        for b in range(B):
            ctx = jnp.concatenate(
                [(a_st[b * H + hh] / l_st[b * H + hh]).astype(jnp.bfloat16)
                 for hh in range(H)], axis=1)
            out_ref[b] = lax.dot_general(ctx, wo_ref[...],
                                         (((1,), (0,)), ((), ())),
                                         preferred_element_type=jnp.float32)

    return pl.pallas_call(
        body,
        out_shape=jax.ShapeDtypeStruct((B, S, Dm), jnp.float32),
        in_specs=[pl.BlockSpec(memory_space=pltpu.VMEM)] * 4,
        out_specs=pl.BlockSpec(memory_space=pltpu.VMEM),
        scratch_shapes=[
            pltpu.VMEM((N_DEV, 2 * B, S, HD), jnp.bfloat16),
            pltpu.SemaphoreType.DMA,
            pltpu.SemaphoreType.DMA((R_HOPS, 2)),
            pltpu.SemaphoreType.DMA((R_HOPS, 2)),
            pltpu.SemaphoreType.DMA((L_HOPS, 2)),
            pltpu.SemaphoreType.DMA((L_HOPS, 2)),
        ],
        compiler_params=pltpu.CompilerParams(collective_id=0),
    )(xb, wqb, kv, wob)
